# Initial kernel scaffold; baseline (speedup 1.0000x reference)
import functools

import jax
import jax.numpy as jnp
from jax import lax
from jax.experimental import pallas as pl
from jax.experimental.pallas import tpu as pltpu
from jax.experimental.pallas import tpu_sc as plsc

NC = 2
NS = 16
NW = NC * NS
K = 128
CPW = 80
EPAD = NW * CPW * K
NPAD = 10240
RPT = NPAD // NS

_mesh = functools.partial(
    plsc.VectorSubcoreMesh,
    core_axis_name="c", subcore_axis_name="s", num_cores=NC, num_subcores=NS,
)


def _deg_body(dst_hbm, zeros_hbm, out_hbm, dst_v, ones_v, acc, sem):
    c = lax.axis_index("c")
    s = lax.axis_index("s")
    wid = s * NC + c
    pltpu.sync_copy(dst_hbm.at[wid], dst_v)
    for i in range(K // 16):
        ones_v[pl.ds(i * 16, 16)] = jnp.ones((16,), jnp.float32)
    pltpu.sync_copy(zeros_hbm.at[pl.ds(s * RPT, RPT)],
                    acc.at[pl.ds(s * RPT, RPT)])
    plsc.subcore_barrier()

    def chunk(j, _):
        pltpu.sync_copy(ones_v, acc.at[dst_v.at[j]], add=True)
        return ()

    lax.fori_loop(0, CPW, chunk, ())
    plsc.subcore_barrier()
    pltpu.sync_copy(acc.at[pl.ds(s * RPT, RPT)],
                    out_hbm.at[c, pl.ds(s * RPT, RPT)])


def _degree(dst3, zeros1):
    return pl.kernel(
        _deg_body,
        out_type=jax.ShapeDtypeStruct((NC, NPAD), jnp.float32),
        mesh=_mesh(),
        scratch_types=[
            pltpu.VMEM((CPW, K), jnp.int32),
            pltpu.VMEM((K,), jnp.float32),
            pltpu.VMEM_SHARED((NPAD,), jnp.float32),
            pltpu.SemaphoreType.DMA,
        ],
    )(dst3, zeros1)


def _seg_body(src_hbm, dst_hbm, h_hbm, zeros_hbm, out_hbm,
              src_v, dst_v, buf0, buf1, acc, sem0, sem1):
    c = lax.axis_index("c")
    s = lax.axis_index("s")
    wid = s * NC + c
    pltpu.sync_copy(src_hbm.at[wid], src_v)
    pltpu.sync_copy(dst_hbm.at[wid], dst_v)
    pltpu.sync_copy(zeros_hbm.at[pl.ds(s * RPT, RPT)],
                    acc.at[pl.ds(s * RPT, RPT)])
    plsc.subcore_barrier()

    bufs = (buf0, buf1)
    sems = (sem0, sem1)
    pltpu.async_copy(h_hbm.at[src_v.at[0]], buf0, sem0)
    pltpu.async_copy(h_hbm.at[src_v.at[1]], buf1, sem1)

    def step(i, _):
        g = i * 2
        for b in range(2):
            j = g + b
            pltpu.make_async_copy(h_hbm.at[src_v.at[j]], bufs[b], sems[b]).wait()
            pltpu.sync_copy(bufs[b], acc.at[dst_v.at[j]], add=True)

            @pl.when(j + 2 < CPW)
            def _():
                pltpu.async_copy(h_hbm.at[src_v.at[j + 2]], bufs[b], sems[b])
        return ()

    lax.fori_loop(0, CPW // 2, step, ())
    plsc.subcore_barrier()
    pltpu.sync_copy(acc.at[pl.ds(s * RPT, RPT)],
                    out_hbm.at[c, pl.ds(s * RPT, RPT)])


def _segment_sum(src3, dst3, h, zeros2, d):
    return pl.kernel(
        _seg_body,
        out_type=jax.ShapeDtypeStruct((NC, NPAD, d), jnp.float32),
        mesh=_mesh(),
        scratch_types=[
            pltpu.VMEM((CPW, K), jnp.int32),
            pltpu.VMEM((CPW, K), jnp.int32),
            pltpu.VMEM((K, d), jnp.float32),
            pltpu.VMEM((K, d), jnp.float32),
            pltpu.VMEM_SHARED((NPAD, d), jnp.float32),
            pltpu.SemaphoreType.DMA,
            pltpu.SemaphoreType.DMA,
        ],
    )(src3, dst3, h, zeros2)


_BR = 512
_GRID = NPAD // _BR


def _mm_scale_body(x_ref, w_ref, dinv_ref, o_ref):
    h = jnp.dot(x_ref[...], w_ref[...], preferred_element_type=jnp.float32)
    o_ref[...] = h * dinv_ref[...]


def _mm_scale(xp, w, dinv):
    d_in, d_out = w.shape
    return pl.pallas_call(
        _mm_scale_body,
        grid=(_GRID,),
        in_specs=[
            pl.BlockSpec((_BR, d_in), lambda i: (i, 0)),
            pl.BlockSpec((d_in, d_out), lambda i: (0, 0)),
            pl.BlockSpec((_BR, 1), lambda i: (i, 0)),
        ],
        out_specs=pl.BlockSpec((_BR, d_out), lambda i: (i, 0)),
        out_shape=jax.ShapeDtypeStruct((NPAD, d_out), jnp.float32),
    )(xp, w, dinv)


def _mid_body(acc_ref, hp_ref, dinv_ref, b1_ref, w2_ref, o_ref):
    z = (acc_ref[0] + acc_ref[1] + hp_ref[...]) * dinv_ref[...] + b1_ref[...]
    z = jnp.maximum(z, 0.0)
    h2 = jnp.dot(z, w2_ref[...], preferred_element_type=jnp.float32)
    o_ref[...] = h2 * dinv_ref[...]


def _mid(acc1, h1p, dinv, b1, w2):
    d_in, d_out = w2.shape
    return pl.pallas_call(
        _mid_body,
        grid=(_GRID,),
        in_specs=[
            pl.BlockSpec((NC, _BR, d_in), lambda i: (0, i, 0)),
            pl.BlockSpec((_BR, d_in), lambda i: (i, 0)),
            pl.BlockSpec((_BR, 1), lambda i: (i, 0)),
            pl.BlockSpec((1, d_in), lambda i: (0, 0)),
            pl.BlockSpec((d_in, d_out), lambda i: (0, 0)),
        ],
        out_specs=pl.BlockSpec((_BR, d_out), lambda i: (i, 0)),
        out_shape=jax.ShapeDtypeStruct((NPAD, d_out), jnp.float32),
    )(acc1, h1p, dinv, b1, w2)


def _post_body(acc_ref, hp_ref, dinv_ref, b2_ref, o_ref):
    t = (acc_ref[0] + acc_ref[1] + hp_ref[...]) * dinv_ref[...] + b2_ref[...]
    m = jnp.max(t, axis=1, keepdims=True)
    e = jnp.exp(t - m)
    lse = jnp.log(jnp.sum(e, axis=1, keepdims=True)) + m
    o_ref[...] = t - lse


def _post(acc2, h2p, dinv, b2):
    d = h2p.shape[1]
    return pl.pallas_call(
        _post_body,
        grid=(_GRID,),
        in_specs=[
            pl.BlockSpec((NC, _BR, d), lambda i: (0, i, 0)),
            pl.BlockSpec((_BR, d), lambda i: (i, 0)),
            pl.BlockSpec((_BR, 1), lambda i: (i, 0)),
            pl.BlockSpec((1, d), lambda i: (0, 0)),
        ],
        out_specs=pl.BlockSpec((_BR, d), lambda i: (i, 0)),
        out_shape=jax.ShapeDtypeStruct((NPAD, d), jnp.float32),
    )(acc2, h2p, dinv, b2)


def kernel(x, edge_index, W1, b1, W2, b2):
    n, d_in = x.shape
    e = edge_index.shape[1]
    d_h = W1.shape[1]
    d_out = W2.shape[1]

    xp = jnp.pad(x, ((0, NPAD - n), (0, 0)))
    pad_e = EPAD - e
    src3 = jnp.concatenate(
        [edge_index[0], jnp.zeros((pad_e,), jnp.int32)]).reshape(NW, CPW, K)
    dst3 = jnp.concatenate(
        [edge_index[1], jnp.full((pad_e,), n, jnp.int32)]).reshape(NW, CPW, K)
    zeros1 = jnp.zeros((NPAD,), jnp.float32)
    zeros_h = jnp.zeros((NPAD, d_h), jnp.float32)
    zeros_o = jnp.zeros((NPAD, d_out), jnp.float32)

    deg2 = _degree(dst3, zeros1)
    dinv = lax.rsqrt(deg2[0] + deg2[1] + 1.0).reshape(NPAD, 1)

    h1p = _mm_scale(xp, W1, dinv)
    acc1 = _segment_sum(src3, dst3, h1p, zeros_h, d_h)
    h2p = _mid(acc1, h1p, dinv, b1.reshape(1, d_h), W2)
    acc2 = _segment_sum(src3, dst3, h2p, zeros_o, d_out)
    out = _post(acc2, h2p, dinv, b2.reshape(1, d_out))
    return out[:n]


# speedup vs baseline: 11.8055x; 11.8055x over previous
import functools

import jax
import jax.numpy as jnp
from jax import lax
from jax.experimental import pallas as pl
from jax.experimental.pallas import tpu as pltpu
from jax.experimental.pallas import tpu_sc as plsc

NC = 2
NS = 16
NW = NC * NS
K = 128
CPW = 80
EPAD = NW * CPW * K
NACC = 10112
RPTA = NACC // NS
DW = 16
DH = 64

_mesh = functools.partial(
    plsc.VectorSubcoreMesh,
    core_axis_name="c", subcore_axis_name="s", num_cores=NC, num_subcores=NS,
)


def _deg_body(dst_hbm, zeros_hbm, out_hbm, dst_v, ones_v, acc):
    c = lax.axis_index("c")
    s = lax.axis_index("s")
    wid = s * NC + c
    pltpu.sync_copy(dst_hbm.at[wid], dst_v)
    for i in range(K):
        ones_v[i] = jnp.ones((DW,), jnp.float32)
    pltpu.sync_copy(zeros_hbm.at[pl.ds(s * RPTA, RPTA)],
                    acc.at[pl.ds(s * RPTA, RPTA)])
    plsc.subcore_barrier()

    def chunk(j, _):
        pltpu.sync_copy(ones_v, acc.at[dst_v.at[j]], add=True)
        return ()

    lax.fori_loop(0, CPW, chunk, ())
    plsc.subcore_barrier()
    pltpu.sync_copy(acc.at[pl.ds(s * RPTA, RPTA)],
                    out_hbm.at[c, pl.ds(s * RPTA, RPTA)])


def _degree(dst3, zeros_deg):
    return pl.kernel(
        _deg_body,
        out_type=jax.ShapeDtypeStruct((NC, NACC, DW), jnp.float32),
        mesh=_mesh(),
        compiler_params=pltpu.CompilerParams(use_tc_tiling_on_sc=False),
        scratch_types=[
            pltpu.VMEM((CPW, K), jnp.int32),
            pltpu.VMEM((K, DW), jnp.float32),
            pltpu.VMEM_SHARED((NACC, DW), jnp.float32),
        ],
    )(dst3, zeros_deg)


def _seg_pass(h_hbm, src_v, dst_v, acc, bufs, sems):
    pltpu.async_copy(h_hbm.at[src_v.at[0]], bufs[0], sems[0])
    pltpu.async_copy(h_hbm.at[src_v.at[1]], bufs[1], sems[1])

    def step(i, _):
        g = i * 2
        for b in range(2):
            j = g + b
            pltpu.make_async_copy(h_hbm.at[src_v.at[j]], bufs[b], sems[b]).wait()
            pltpu.sync_copy(bufs[b], acc.at[dst_v.at[j]], add=True)

            @pl.when(j + 2 < CPW)
            def _():
                pltpu.async_copy(h_hbm.at[src_v.at[j + 2]], bufs[b], sems[b])
        return ()

    lax.fori_loop(0, CPW // 2, step, ())


def _zero_acc(zeros_hbm, acc, s):
    pltpu.sync_copy(zeros_hbm.at[pl.ds(s * RPTA, RPTA)],
                    acc.at[pl.ds(s * RPTA, RPTA)])


def _seg1_body(src_hbm, dst_hbm, hl_hbm, hr_hbm, zeros_hbm, out_hbm,
               src_v, dst_v, buf0, buf1, acc, sem0, sem1):
    c = lax.axis_index("c")
    s = lax.axis_index("s")
    wid = s * NC + c
    pltpu.sync_copy(src_hbm.at[wid], src_v)
    pltpu.sync_copy(dst_hbm.at[wid], dst_v)
    bufs, sems = (buf0, buf1), (sem0, sem1)
    for half, h_hbm in enumerate((hl_hbm, hr_hbm)):
        _zero_acc(zeros_hbm, acc, s)
        plsc.subcore_barrier()
        _seg_pass(h_hbm, src_v, dst_v, acc, bufs, sems)
        plsc.subcore_barrier()
        pltpu.sync_copy(acc.at[pl.ds(s * RPTA, RPTA)],
                        out_hbm.at[c, half, pl.ds(s * RPTA, RPTA)])
        plsc.subcore_barrier()


def _segment_sum2(src3, dst3, hl, hr, zeros2):
    return pl.kernel(
        _seg1_body,
        out_type=jax.ShapeDtypeStruct((NC, 2, NACC, DH), jnp.float32),
        mesh=_mesh(),
        compiler_params=pltpu.CompilerParams(use_tc_tiling_on_sc=False),
        scratch_types=[
            pltpu.VMEM((CPW, K), jnp.int32),
            pltpu.VMEM((CPW, K), jnp.int32),
            pltpu.VMEM((K, DH), jnp.float32),
            pltpu.VMEM((K, DH), jnp.float32),
            pltpu.VMEM_SHARED((NACC, DH), jnp.float32),
            pltpu.SemaphoreType.DMA,
            pltpu.SemaphoreType.DMA,
        ],
    )(src3, dst3, hl, hr, zeros2)


def _seg2_body(src_hbm, dst_hbm, h_hbm, zeros_hbm, out_hbm,
               src_v, dst_v, buf0, buf1, acc, sem0, sem1):
    c = lax.axis_index("c")
    s = lax.axis_index("s")
    wid = s * NC + c
    pltpu.sync_copy(src_hbm.at[wid], src_v)
    pltpu.sync_copy(dst_hbm.at[wid], dst_v)
    _zero_acc(zeros_hbm, acc, s)
    plsc.subcore_barrier()
    _seg_pass(h_hbm, src_v, dst_v, acc, (buf0, buf1), (sem0, sem1))
    plsc.subcore_barrier()
    pltpu.sync_copy(acc.at[pl.ds(s * RPTA, RPTA)],
                    out_hbm.at[c, pl.ds(s * RPTA, RPTA)])


def _segment_sum(src3, dst3, h, zeros2):
    return pl.kernel(
        _seg2_body,
        out_type=jax.ShapeDtypeStruct((NC, NACC, DH), jnp.float32),
        mesh=_mesh(),
        compiler_params=pltpu.CompilerParams(use_tc_tiling_on_sc=False),
        scratch_types=[
            pltpu.VMEM((CPW, K), jnp.int32),
            pltpu.VMEM((CPW, K), jnp.int32),
            pltpu.VMEM((K, DH), jnp.float32),
            pltpu.VMEM((K, DH), jnp.float32),
            pltpu.VMEM_SHARED((NACC, DH), jnp.float32),
            pltpu.SemaphoreType.DMA,
            pltpu.SemaphoreType.DMA,
        ],
    )(src3, dst3, h, zeros2)


_BR = 632
_GRID = NACC // _BR


def _mm_scale_body(x_ref, w_ref, dinv_ref, ol_ref, or_ref):
    h = jnp.dot(x_ref[...], w_ref[...], preferred_element_type=jnp.float32)
    hs = h * dinv_ref[...]
    ol_ref[...] = hs[:, :DH]
    or_ref[...] = hs[:, DH:]


def _mm_scale(xp, w, dinv):
    d_in, d_out = w.shape
    return pl.pallas_call(
        _mm_scale_body,
        grid=(_GRID,),
        in_specs=[
            pl.BlockSpec((_BR, d_in), lambda i: (i, 0)),
            pl.BlockSpec((d_in, d_out), lambda i: (0, 0)),
            pl.BlockSpec((_BR, 1), lambda i: (i, 0)),
        ],
        out_specs=[
            pl.BlockSpec((_BR, DH), lambda i: (i, 0)),
            pl.BlockSpec((_BR, DH), lambda i: (i, 0)),
        ],
        out_shape=[
            jax.ShapeDtypeStruct((NACC, DH), jnp.float32),
            jax.ShapeDtypeStruct((NACC, DH), jnp.float32),
        ],
    )(xp, w, dinv)


def _mid_body(acc_ref, hl_ref, hr_ref, dinv_ref, b1_ref, w2_ref, o_ref):
    al = acc_ref[0, 0] + acc_ref[1, 0] + hl_ref[...]
    ar = acc_ref[0, 1] + acc_ref[1, 1] + hr_ref[...]
    z = jnp.concatenate([al, ar], axis=1) * dinv_ref[...] + b1_ref[...]
    z = jnp.maximum(z, 0.0)
    h2 = jnp.dot(z, w2_ref[...], preferred_element_type=jnp.float32)
    o_ref[...] = h2 * dinv_ref[...]


def _mid(acc1, hl, hr, dinv, b1, w2):
    d_in, d_out = w2.shape
    return pl.pallas_call(
        _mid_body,
        grid=(_GRID,),
        in_specs=[
            pl.BlockSpec((NC, 2, _BR, DH), lambda i: (0, 0, i, 0)),
            pl.BlockSpec((_BR, DH), lambda i: (i, 0)),
            pl.BlockSpec((_BR, DH), lambda i: (i, 0)),
            pl.BlockSpec((_BR, 1), lambda i: (i, 0)),
            pl.BlockSpec((1, d_in), lambda i: (0, 0)),
            pl.BlockSpec((d_in, d_out), lambda i: (0, 0)),
        ],
        out_specs=pl.BlockSpec((_BR, d_out), lambda i: (i, 0)),
        out_shape=jax.ShapeDtypeStruct((NACC, d_out), jnp.float32),
    )(acc1, hl, hr, dinv, b1, w2)


def _post_body(acc_ref, hp_ref, dinv_ref, b2_ref, o_ref):
    t = (acc_ref[0] + acc_ref[1] + hp_ref[...]) * dinv_ref[...] + b2_ref[...]
    m = jnp.max(t, axis=1, keepdims=True)
    e = jnp.exp(t - m)
    lse = jnp.log(jnp.sum(e, axis=1, keepdims=True)) + m
    o_ref[...] = t - lse


def _post(acc2, h2p, dinv, b2):
    d = h2p.shape[1]
    return pl.pallas_call(
        _post_body,
        grid=(_GRID,),
        in_specs=[
            pl.BlockSpec((NC, _BR, d), lambda i: (0, i, 0)),
            pl.BlockSpec((_BR, d), lambda i: (i, 0)),
            pl.BlockSpec((_BR, 1), lambda i: (i, 0)),
            pl.BlockSpec((1, d), lambda i: (0, 0)),
        ],
        out_specs=pl.BlockSpec((_BR, d), lambda i: (i, 0)),
        out_shape=jax.ShapeDtypeStruct((NACC, d), jnp.float32),
    )(acc2, h2p, dinv, b2)


def kernel(x, edge_index, W1, b1, W2, b2):
    n, d_in = x.shape
    e = edge_index.shape[1]
    d_h = W1.shape[1]
    d_out = W2.shape[1]

    xp = jnp.pad(x, ((0, NACC - n), (0, 0)))
    pad_e = EPAD - e
    src3 = jnp.concatenate(
        [edge_index[0], jnp.zeros((pad_e,), jnp.int32)]).reshape(NW, CPW, K)
    dst3 = jnp.concatenate(
        [edge_index[1], jnp.full((pad_e,), n, jnp.int32)]).reshape(NW, CPW, K)
    zeros_deg = jnp.zeros((NACC, DW), jnp.float32)
    zeros2 = jnp.zeros((NACC, DH), jnp.float32)

    deg_parts = _degree(dst3, zeros_deg)
    deg = deg_parts[0, :, 0] + deg_parts[1, :, 0]
    dinv = lax.rsqrt(deg + 1.0).reshape(NACC, 1)

    h1l, h1r = _mm_scale(xp, W1, dinv)
    acc1 = _segment_sum2(src3, dst3, h1l, h1r, zeros2)
    h2p = _mid(acc1, h1l, h1r, dinv, b1.reshape(1, d_h), W2)
    acc2 = _segment_sum(src3, dst3, h2p, zeros2)
    out = _post(acc2, h2p, dinv, b2.reshape(1, d_out))
    return out[:n]


# speedup vs baseline: 12.4999x; 1.0588x over previous
import functools

import jax
import jax.numpy as jnp
from jax import lax
from jax.experimental import pallas as pl
from jax.experimental.pallas import tpu as pltpu
from jax.experimental.pallas import tpu_sc as plsc

NC = 2
NS = 16
NW = NC * NS
K = 128
CPW = 80
EPAD = NW * CPW * K
TCH = NW * CPW
CPW0 = 118
CPW1 = 42
NACC = 10112
RPTA = NACC // NS
DW = 16
DH = 64

_mesh = functools.partial(
    plsc.VectorSubcoreMesh,
    core_axis_name="c", subcore_axis_name="s", num_cores=NC, num_subcores=NS,
)


def _deg_body(dst_hbm, zeros_hbm, out_hbm, dst_v, ones_v, acc):
    c = lax.axis_index("c")
    s = lax.axis_index("s")
    wid = s * NC + c
    pltpu.sync_copy(dst_hbm.at[wid], dst_v)
    for i in range(K):
        ones_v[i] = jnp.ones((DW,), jnp.float32)
    pltpu.sync_copy(zeros_hbm.at[pl.ds(s * RPTA, RPTA)],
                    acc.at[pl.ds(s * RPTA, RPTA)])
    plsc.subcore_barrier()

    def chunk(j, _):
        pltpu.sync_copy(ones_v, acc.at[dst_v.at[j]], add=True)
        return ()

    lax.fori_loop(0, CPW, chunk, ())
    plsc.subcore_barrier()
    pltpu.sync_copy(acc.at[pl.ds(s * RPTA, RPTA)],
                    out_hbm.at[c, pl.ds(s * RPTA, RPTA)])


def _degree(dst3, zeros_deg):
    return pl.kernel(
        _deg_body,
        out_type=jax.ShapeDtypeStruct((NC, NACC, DW), jnp.float32),
        mesh=_mesh(),
        compiler_params=pltpu.CompilerParams(use_tc_tiling_on_sc=False),
        scratch_types=[
            pltpu.VMEM((CPW, K), jnp.int32),
            pltpu.VMEM((K, DW), jnp.float32),
            pltpu.VMEM_SHARED((NACC, DW), jnp.float32),
        ],
    )(dst3, zeros_deg)


def _seg_pass(h_hbm, src_v, dst_v, acc, bufs, sems, cnt):
    pltpu.async_copy(h_hbm.at[src_v.at[0]], bufs[0], sems[0])
    pltpu.async_copy(h_hbm.at[src_v.at[1]], bufs[1], sems[1])

    def step(i, _):
        g = i * 2
        for b in range(2):
            j = g + b
            pltpu.make_async_copy(h_hbm.at[src_v.at[j]], bufs[b], sems[b]).wait()
            pltpu.sync_copy(bufs[b], acc.at[dst_v.at[j]], add=True)

            @pl.when(j + 2 < cnt)
            def _():
                pltpu.async_copy(h_hbm.at[src_v.at[j + 2]], bufs[b], sems[b])
        return ()

    lax.fori_loop(0, cnt // 2, step, ())


def _zero_acc(zeros_hbm, acc, s):
    pltpu.sync_copy(zeros_hbm.at[pl.ds(s * RPTA, RPTA)],
                    acc.at[pl.ds(s * RPTA, RPTA)])


def _stage_indices(src_hbm, dst_hbm, src_v, dst_v, c, s):
    @pl.when(c == 0)
    def _():
        pltpu.sync_copy(src_hbm.at[pl.ds(s * CPW0, CPW0)],
                        src_v.at[pl.ds(0, CPW0)])
        pltpu.sync_copy(dst_hbm.at[pl.ds(s * CPW0, CPW0)],
                        dst_v.at[pl.ds(0, CPW0)])

    @pl.when(c == 1)
    def _():
        base = NS * CPW0 + s * CPW1
        pltpu.sync_copy(src_hbm.at[pl.ds(base, CPW1)],
                        src_v.at[pl.ds(0, CPW1)])
        pltpu.sync_copy(dst_hbm.at[pl.ds(base, CPW1)],
                        dst_v.at[pl.ds(0, CPW1)])
    return jnp.where(c == 0, CPW0, CPW1)


def _seg1_body(src_hbm, dst_hbm, hl_hbm, hr_hbm, zeros_hbm, out_hbm,
               src_v, dst_v, buf0, buf1, acc, sem0, sem1):
    c = lax.axis_index("c")
    s = lax.axis_index("s")
    cnt = _stage_indices(src_hbm, dst_hbm, src_v, dst_v, c, s)
    bufs, sems = (buf0, buf1), (sem0, sem1)
    for half, h_hbm in enumerate((hl_hbm, hr_hbm)):
        _zero_acc(zeros_hbm, acc, s)
        plsc.subcore_barrier()
        _seg_pass(h_hbm, src_v, dst_v, acc, bufs, sems, cnt)
        plsc.subcore_barrier()
        pltpu.sync_copy(acc.at[pl.ds(s * RPTA, RPTA)],
                        out_hbm.at[c, half, pl.ds(s * RPTA, RPTA)])
        plsc.subcore_barrier()


def _segment_sum2(src2, dst2, hl, hr, zeros2):
    return pl.kernel(
        _seg1_body,
        out_type=jax.ShapeDtypeStruct((NC, 2, NACC, DH), jnp.float32),
        mesh=_mesh(),
        compiler_params=pltpu.CompilerParams(use_tc_tiling_on_sc=False),
        scratch_types=[
            pltpu.VMEM((CPW0, K), jnp.int32),
            pltpu.VMEM((CPW0, K), jnp.int32),
            pltpu.VMEM((K, DH), jnp.float32),
            pltpu.VMEM((K, DH), jnp.float32),
            pltpu.VMEM_SHARED((NACC, DH), jnp.float32),
            pltpu.SemaphoreType.DMA,
            pltpu.SemaphoreType.DMA,
        ],
    )(src2, dst2, hl, hr, zeros2)


def _seg2_body(src_hbm, dst_hbm, h_hbm, zeros_hbm, out_hbm,
               src_v, dst_v, buf0, buf1, acc, sem0, sem1):
    c = lax.axis_index("c")
    s = lax.axis_index("s")
    cnt = _stage_indices(src_hbm, dst_hbm, src_v, dst_v, c, s)
    _zero_acc(zeros_hbm, acc, s)
    plsc.subcore_barrier()
    _seg_pass(h_hbm, src_v, dst_v, acc, (buf0, buf1), (sem0, sem1), cnt)
    plsc.subcore_barrier()
    pltpu.sync_copy(acc.at[pl.ds(s * RPTA, RPTA)],
                    out_hbm.at[c, pl.ds(s * RPTA, RPTA)])


def _segment_sum(src2, dst2, h, zeros2):
    return pl.kernel(
        _seg2_body,
        out_type=jax.ShapeDtypeStruct((NC, NACC, DH), jnp.float32),
        mesh=_mesh(),
        compiler_params=pltpu.CompilerParams(use_tc_tiling_on_sc=False),
        scratch_types=[
            pltpu.VMEM((CPW0, K), jnp.int32),
            pltpu.VMEM((CPW0, K), jnp.int32),
            pltpu.VMEM((K, DH), jnp.float32),
            pltpu.VMEM((K, DH), jnp.float32),
            pltpu.VMEM_SHARED((NACC, DH), jnp.float32),
            pltpu.SemaphoreType.DMA,
            pltpu.SemaphoreType.DMA,
        ],
    )(src2, dst2, h, zeros2)


_BR = 632
_GRID = NACC // _BR


def _mm_scale_body(x_ref, w_ref, dinv_ref, ol_ref, or_ref):
    h = jnp.dot(x_ref[...], w_ref[...], preferred_element_type=jnp.float32)
    hs = h * dinv_ref[...]
    ol_ref[...] = hs[:, :DH]
    or_ref[...] = hs[:, DH:]


def _mm_scale(xp, w, dinv):
    d_in, d_out = w.shape
    return pl.pallas_call(
        _mm_scale_body,
        grid=(_GRID,),
        in_specs=[
            pl.BlockSpec((_BR, d_in), lambda i: (i, 0)),
            pl.BlockSpec((d_in, d_out), lambda i: (0, 0)),
            pl.BlockSpec((_BR, 1), lambda i: (i, 0)),
        ],
        out_specs=[
            pl.BlockSpec((_BR, DH), lambda i: (i, 0)),
            pl.BlockSpec((_BR, DH), lambda i: (i, 0)),
        ],
        out_shape=[
            jax.ShapeDtypeStruct((NACC, DH), jnp.float32),
            jax.ShapeDtypeStruct((NACC, DH), jnp.float32),
        ],
    )(xp, w, dinv)


def _mid_body(acc_ref, hl_ref, hr_ref, dinv_ref, b1_ref, w2_ref, o_ref):
    al = acc_ref[0, 0] + acc_ref[1, 0] + hl_ref[...]
    ar = acc_ref[0, 1] + acc_ref[1, 1] + hr_ref[...]
    z = jnp.concatenate([al, ar], axis=1) * dinv_ref[...] + b1_ref[...]
    z = jnp.maximum(z, 0.0)
    h2 = jnp.dot(z, w2_ref[...], preferred_element_type=jnp.float32)
    o_ref[...] = h2 * dinv_ref[...]


def _mid(acc1, hl, hr, dinv, b1, w2):
    d_in, d_out = w2.shape
    return pl.pallas_call(
        _mid_body,
        grid=(_GRID,),
        in_specs=[
            pl.BlockSpec((NC, 2, _BR, DH), lambda i: (0, 0, i, 0)),
            pl.BlockSpec((_BR, DH), lambda i: (i, 0)),
            pl.BlockSpec((_BR, DH), lambda i: (i, 0)),
            pl.BlockSpec((_BR, 1), lambda i: (i, 0)),
            pl.BlockSpec((1, d_in), lambda i: (0, 0)),
            pl.BlockSpec((d_in, d_out), lambda i: (0, 0)),
        ],
        out_specs=pl.BlockSpec((_BR, d_out), lambda i: (i, 0)),
        out_shape=jax.ShapeDtypeStruct((NACC, d_out), jnp.float32),
    )(acc1, hl, hr, dinv, b1, w2)


def _post_body(acc_ref, hp_ref, dinv_ref, b2_ref, o_ref):
    t = (acc_ref[0] + acc_ref[1] + hp_ref[...]) * dinv_ref[...] + b2_ref[...]
    m = jnp.max(t, axis=1, keepdims=True)
    e = jnp.exp(t - m)
    lse = jnp.log(jnp.sum(e, axis=1, keepdims=True)) + m
    o_ref[...] = t - lse


def _post(acc2, h2p, dinv, b2):
    d = h2p.shape[1]
    return pl.pallas_call(
        _post_body,
        grid=(_GRID,),
        in_specs=[
            pl.BlockSpec((NC, _BR, d), lambda i: (0, i, 0)),
            pl.BlockSpec((_BR, d), lambda i: (i, 0)),
            pl.BlockSpec((_BR, 1), lambda i: (i, 0)),
            pl.BlockSpec((1, d), lambda i: (0, 0)),
        ],
        out_specs=pl.BlockSpec((_BR, d), lambda i: (i, 0)),
        out_shape=jax.ShapeDtypeStruct((NACC, d), jnp.float32),
    )(acc2, h2p, dinv, b2)


def kernel(x, edge_index, W1, b1, W2, b2):
    n, d_in = x.shape
    e = edge_index.shape[1]
    d_h = W1.shape[1]
    d_out = W2.shape[1]

    xp = jnp.pad(x, ((0, NACC - n), (0, 0)))
    pad_e = EPAD - e
    src3 = jnp.concatenate(
        [edge_index[0], jnp.zeros((pad_e,), jnp.int32)]).reshape(NW, CPW, K)
    dst3 = jnp.concatenate(
        [edge_index[1], jnp.full((pad_e,), n, jnp.int32)]).reshape(NW, CPW, K)
    zeros_deg = jnp.zeros((NACC, DW), jnp.float32)
    zeros2 = jnp.zeros((NACC, DH), jnp.float32)

    deg_parts = _degree(dst3, zeros_deg)
    deg = deg_parts[0, :, 0] + deg_parts[1, :, 0]
    dinv = lax.rsqrt(deg + 1.0).reshape(NACC, 1)

    h1l, h1r = _mm_scale(xp, W1, dinv)
    src2 = src3.reshape(TCH, K)
    dst2 = dst3.reshape(TCH, K)
    acc1 = _segment_sum2(src2, dst2, h1l, h1r, zeros2)
    h2p = _mid(acc1, h1l, h1r, dinv, b1.reshape(1, d_h), W2)
    acc2 = _segment_sum(src2, dst2, h2p, zeros2)
    out = _post(acc2, h2p, dinv, b2.reshape(1, d_out))
    return out[:n]


# speedup vs baseline: 14.3815x; 1.1505x over previous
import functools

import jax
import jax.numpy as jnp
from jax import lax
from jax.experimental import pallas as pl
from jax.experimental.pallas import tpu as pltpu
from jax.experimental.pallas import tpu_sc as plsc

NC = 2
NS = 16
NW = NC * NS
K = 128
CPW = 80
EPAD = NW * CPW * K
TCH = NW * CPW
CPT = TCH // NS
ZR = 79
NACC = 10112
RPTA = NACC // NS
DW = 16
DH = 64

_mesh = functools.partial(
    plsc.VectorSubcoreMesh,
    core_axis_name="c", subcore_axis_name="s", num_cores=NC, num_subcores=NS,
)


def _deg_body(dst_hbm, zeros_hbm, out_hbm, dst_v, ones_v, acc):
    c = lax.axis_index("c")
    s = lax.axis_index("s")
    wid = s * NC + c
    pltpu.sync_copy(dst_hbm.at[wid], dst_v)
    for i in range(K):
        ones_v[i] = jnp.ones((DW,), jnp.float32)
    pltpu.sync_copy(zeros_hbm.at[pl.ds(s * RPTA, RPTA)],
                    acc.at[pl.ds(s * RPTA, RPTA)])
    plsc.subcore_barrier()

    def chunk(j, _):
        pltpu.sync_copy(ones_v, acc.at[dst_v.at[j]], add=True)
        return ()

    lax.fori_loop(0, CPW, chunk, ())
    plsc.subcore_barrier()
    pltpu.sync_copy(acc.at[pl.ds(s * RPTA, RPTA)],
                    out_hbm.at[c, pl.ds(s * RPTA, RPTA)])


def _degree(dst3, zeros_deg):
    return pl.kernel(
        _deg_body,
        out_type=jax.ShapeDtypeStruct((NC, NACC, DW), jnp.float32),
        mesh=_mesh(),
        compiler_params=pltpu.CompilerParams(use_tc_tiling_on_sc=False),
        scratch_types=[
            pltpu.VMEM((CPW, K), jnp.int32),
            pltpu.VMEM((K, DW), jnp.float32),
            pltpu.VMEM_SHARED((NACC, DW), jnp.float32),
        ],
    )(dst3, zeros_deg)


def _seg_pass(h_hbm, src_v, dst_v, acc, bufs, sems, cnt):
    pltpu.async_copy(h_hbm.at[src_v.at[0]], bufs[0], sems[0])
    pltpu.async_copy(h_hbm.at[src_v.at[1]], bufs[1], sems[1])

    def step(i, _):
        g = i * 2
        for b in range(2):
            j = g + b
            pltpu.make_async_copy(h_hbm.at[src_v.at[j]], bufs[b], sems[b]).wait()
            pltpu.sync_copy(bufs[b], acc.at[dst_v.at[j]], add=True)

            @pl.when(j + 2 < cnt)
            def _():
                pltpu.async_copy(h_hbm.at[src_v.at[j + 2]], bufs[b], sems[b])
        return ()

    lax.fori_loop(0, cnt // 2, step, ())


def _zero_acc(zeros_hbm, acc, s):
    pltpu.sync_copy(zeros_hbm.at[pl.ds(s * RPTA, RPTA)],
                    acc.at[pl.ds(s * RPTA, RPTA)])


def _fill_zeros(zbuf):
    for i in range(ZR):
        for k in range(DH // 16):
            zbuf[i, pl.ds(k * 16, 16)] = jnp.zeros((16,), jnp.float32)


def _zero_acc_local(zbuf, acc, s):
    for k in range(RPTA // ZR):
        pltpu.sync_copy(zbuf, acc.at[pl.ds(s * RPTA + k * ZR, ZR)])


def _stage_all(src_hbm, dst_hbm, src_v, dst_v, s):
    pltpu.sync_copy(src_hbm.at[pl.ds(s * CPT, CPT)], src_v)
    pltpu.sync_copy(dst_hbm.at[pl.ds(s * CPT, CPT)], dst_v)


def _seg_pass(h_hbm, src_v, dst_v, acc, bufs, sems):
    pltpu.async_copy(h_hbm.at[src_v.at[0]], bufs[0], sems[0])
    pltpu.async_copy(h_hbm.at[src_v.at[1]], bufs[1], sems[1])

    def step(i, _):
        g = i * 2
        for b in range(2):
            j = g + b
            pltpu.make_async_copy(h_hbm.at[src_v.at[j]], bufs[b], sems[b]).wait()
            pltpu.sync_copy(bufs[b], acc.at[dst_v.at[j]], add=True)

            @pl.when(j + 2 < CPT)
            def _():
                pltpu.async_copy(h_hbm.at[src_v.at[j + 2]], bufs[b], sems[b])
        return ()

    lax.fori_loop(0, CPT // 2, step, ())


def _seg1_body(src_hbm, dst_hbm, hl_hbm, hr_hbm, zeros_hbm, out_hbm,
               src_v, dst_v, buf0, buf1, zbuf, acc, sem0, sem1):
    c = lax.axis_index("c")
    s = lax.axis_index("s")
    _stage_all(src_hbm, dst_hbm, src_v, dst_v, s)
    _fill_zeros(zbuf)
    _zero_acc_local(zbuf, acc, s)
    plsc.subcore_barrier()

    @pl.when(c == 0)
    def _():
        _seg_pass(hl_hbm, src_v, dst_v, acc, (buf0, buf1), (sem0, sem1))

    @pl.when(c == 1)
    def _():
        _seg_pass(hr_hbm, src_v, dst_v, acc, (buf0, buf1), (sem0, sem1))

    plsc.subcore_barrier()
    pltpu.sync_copy(acc.at[pl.ds(s * RPTA, RPTA)],
                    out_hbm.at[c, pl.ds(s * RPTA, RPTA)])


def _segment_sum2(src2, dst2, hl, hr, zeros2):
    return pl.kernel(
        _seg1_body,
        out_type=jax.ShapeDtypeStruct((NC, NACC, DH), jnp.float32),
        mesh=_mesh(),
        compiler_params=pltpu.CompilerParams(use_tc_tiling_on_sc=False),
        scratch_types=[
            pltpu.VMEM((CPT, K), jnp.int32),
            pltpu.VMEM((CPT, K), jnp.int32),
            pltpu.VMEM((K, DH), jnp.float32),
            pltpu.VMEM((K, DH), jnp.float32),
            pltpu.VMEM((ZR, DH), jnp.float32),
            pltpu.VMEM_SHARED((NACC, DH), jnp.float32),
            pltpu.SemaphoreType.DMA,
            pltpu.SemaphoreType.DMA,
        ],
    )(src2, dst2, hl, hr, zeros2)


def _seg2_body(src_hbm, dst_hbm, h_hbm, zeros_hbm, out_hbm,
               src_v, dst_v, buf0, buf1, zbuf, acc, sem0, sem1):
    c = lax.axis_index("c")
    s = lax.axis_index("s")

    @pl.when(c == 0)
    def _():
        _stage_all(src_hbm, dst_hbm, src_v, dst_v, s)
        _fill_zeros(zbuf)
        _zero_acc_local(zbuf, acc, s)
        plsc.subcore_barrier()
        _seg_pass(h_hbm, src_v, dst_v, acc, (buf0, buf1), (sem0, sem1))
        plsc.subcore_barrier()
        pltpu.sync_copy(acc.at[pl.ds(s * RPTA, RPTA)],
                        out_hbm.at[pl.ds(s * RPTA, RPTA)])


def _segment_sum(src2, dst2, h, zeros2):
    return pl.kernel(
        _seg2_body,
        out_type=jax.ShapeDtypeStruct((NACC, DH), jnp.float32),
        mesh=_mesh(),
        compiler_params=pltpu.CompilerParams(use_tc_tiling_on_sc=False),
        scratch_types=[
            pltpu.VMEM((CPT, K), jnp.int32),
            pltpu.VMEM((CPT, K), jnp.int32),
            pltpu.VMEM((K, DH), jnp.float32),
            pltpu.VMEM((K, DH), jnp.float32),
            pltpu.VMEM((ZR, DH), jnp.float32),
            pltpu.VMEM_SHARED((NACC, DH), jnp.float32),
            pltpu.SemaphoreType.DMA,
            pltpu.SemaphoreType.DMA,
        ],
    )(src2, dst2, h, zeros2)


_BR = 632
_GRID = NACC // _BR


def _mm_scale_body(x_ref, w_ref, dinv_ref, ol_ref, or_ref):
    h = jnp.dot(x_ref[...], w_ref[...], preferred_element_type=jnp.float32)
    hs = h * dinv_ref[...]
    ol_ref[...] = hs[:, :DH]
    or_ref[...] = hs[:, DH:]


def _mm_scale(xp, w, dinv):
    d_in, d_out = w.shape
    return pl.pallas_call(
        _mm_scale_body,
        grid=(_GRID,),
        in_specs=[
            pl.BlockSpec((_BR, d_in), lambda i: (i, 0)),
            pl.BlockSpec((d_in, d_out), lambda i: (0, 0)),
            pl.BlockSpec((_BR, 1), lambda i: (i, 0)),
        ],
        out_specs=[
            pl.BlockSpec((_BR, DH), lambda i: (i, 0)),
            pl.BlockSpec((_BR, DH), lambda i: (i, 0)),
        ],
        out_shape=[
            jax.ShapeDtypeStruct((NACC, DH), jnp.float32),
            jax.ShapeDtypeStruct((NACC, DH), jnp.float32),
        ],
    )(xp, w, dinv)


def _mid_body(acc_ref, hl_ref, hr_ref, dinv_ref, b1_ref, w2_ref, o_ref):
    al = acc_ref[0] + hl_ref[...]
    ar = acc_ref[1] + hr_ref[...]
    z = jnp.concatenate([al, ar], axis=1) * dinv_ref[...] + b1_ref[...]
    z = jnp.maximum(z, 0.0)
    h2 = jnp.dot(z, w2_ref[...], preferred_element_type=jnp.float32)
    o_ref[...] = h2 * dinv_ref[...]


def _mid(acc1, hl, hr, dinv, b1, w2):
    d_in, d_out = w2.shape
    return pl.pallas_call(
        _mid_body,
        grid=(_GRID,),
        in_specs=[
            pl.BlockSpec((NC, _BR, DH), lambda i: (0, i, 0)),
            pl.BlockSpec((_BR, DH), lambda i: (i, 0)),
            pl.BlockSpec((_BR, DH), lambda i: (i, 0)),
            pl.BlockSpec((_BR, 1), lambda i: (i, 0)),
            pl.BlockSpec((1, d_in), lambda i: (0, 0)),
            pl.BlockSpec((d_in, d_out), lambda i: (0, 0)),
        ],
        out_specs=pl.BlockSpec((_BR, d_out), lambda i: (i, 0)),
        out_shape=jax.ShapeDtypeStruct((NACC, d_out), jnp.float32),
    )(acc1, hl, hr, dinv, b1, w2)


def _post_body(acc_ref, hp_ref, dinv_ref, b2_ref, o_ref):
    t = (acc_ref[...] + hp_ref[...]) * dinv_ref[...] + b2_ref[...]
    m = jnp.max(t, axis=1, keepdims=True)
    e = jnp.exp(t - m)
    lse = jnp.log(jnp.sum(e, axis=1, keepdims=True)) + m
    o_ref[...] = t - lse


def _post(acc2, h2p, dinv, b2):
    d = h2p.shape[1]
    return pl.pallas_call(
        _post_body,
        grid=(_GRID,),
        in_specs=[
            pl.BlockSpec((_BR, d), lambda i: (i, 0)),
            pl.BlockSpec((_BR, d), lambda i: (i, 0)),
            pl.BlockSpec((_BR, 1), lambda i: (i, 0)),
            pl.BlockSpec((1, d), lambda i: (0, 0)),
        ],
        out_specs=pl.BlockSpec((_BR, d), lambda i: (i, 0)),
        out_shape=jax.ShapeDtypeStruct((NACC, d), jnp.float32),
    )(acc2, h2p, dinv, b2)


def kernel(x, edge_index, W1, b1, W2, b2):
    n, d_in = x.shape
    e = edge_index.shape[1]
    d_h = W1.shape[1]
    d_out = W2.shape[1]

    xp = jnp.pad(x, ((0, NACC - n), (0, 0)))
    pad_e = EPAD - e
    src3 = jnp.concatenate(
        [edge_index[0], jnp.zeros((pad_e,), jnp.int32)]).reshape(NW, CPW, K)
    dst3 = jnp.concatenate(
        [edge_index[1], jnp.full((pad_e,), n, jnp.int32)]).reshape(NW, CPW, K)
    zeros_deg = jnp.zeros((NACC, DW), jnp.float32)
    zeros2 = jnp.zeros((NACC, DH), jnp.float32)

    deg_parts = _degree(dst3, zeros_deg)
    deg = deg_parts[0, :, 0] + deg_parts[1, :, 0]
    dinv = lax.rsqrt(deg + 1.0).reshape(NACC, 1)

    h1l, h1r = _mm_scale(xp, W1, dinv)
    src2 = src3.reshape(TCH, K)
    dst2 = dst3.reshape(TCH, K)
    acc1 = _segment_sum2(src2, dst2, h1l, h1r, zeros2)
    h2p = _mid(acc1, h1l, h1r, dinv, b1.reshape(1, d_h), W2)
    acc2 = _segment_sum(src2, dst2, h2p, zeros2)
    out = _post(acc2, h2p, dinv, b2.reshape(1, d_out))
    return out[:n]
